# baseline (speedup 1.0000x reference)
import functools

import jax
import jax.numpy as jnp
from jax import lax
from jax.experimental import pallas as pl
from jax.experimental.pallas import tpu as pltpu
from jax.experimental.pallas import tpu_sc as plsc

_NC = 2
_NS = 16
_G = 128
_BLK = 2000


_C = 40
_NB = 5


@functools.lru_cache(maxsize=None)
def _sc_scatter_fn(N, H, E, with_deg):
    NT = _NC * _NS
    EPT = E // NT
    assert E % NT == 0 and EPT % _C == 0
    NCH = EPT // _C
    assert NCH % _NB == 0
    SR = (N // _NS) // 8 * 8
    EXTRA = N - SR * _NS
    assert EXTRA % 16 == 0 and SR % 16 == 0
    NZ = SR // 16

    mesh = plsc.VectorSubcoreMesh(core_axis_name="c", subcore_axis_name="s",
                                  num_cores=_NC, num_subcores=_NS)

    out_type = [jax.ShapeDtypeStruct((N, H), jnp.float32),
                jax.ShapeDtypeStruct((N, H), jnp.float32)]
    scratch = [
        pltpu.VMEM_SHARED((N, H), jnp.float32),
        pltpu.VMEM((EPT,), jnp.int32),
        pltpu.VMEM((16, H), jnp.float32),
    ] + [pltpu.VMEM((_C, H), jnp.float32) for _ in range(_NB)] \
      + [pltpu.VMEM((_C,), jnp.int32) for _ in range(_NB)] \
      + [pltpu.SemaphoreType.DMA for _ in range(3 * _NB + 1)]
    if with_deg:
        out_type += [jax.ShapeDtypeStruct((N,), jnp.float32),
                     jax.ShapeDtypeStruct((N,), jnp.float32)]
        scratch += [
            pltpu.VMEM_SHARED((N,), jnp.float32),
            pltpu.VMEM((SR + 16,), jnp.float32),
            pltpu.VMEM((((_C + 15) // 16) * 16,), jnp.float32),
        ]

    def body(h_hbm, src_hbm, dst_hbm, out0, out1, *rest):
        if with_deg:
            deg0, deg1 = rest[0], rest[1]
            rest = rest[2:]
            deg_sh, zb1, ones = rest[-3:]
        acc_sh, idx_s, zbuf = rest[0], rest[1], rest[2]
        rows = rest[3:3 + _NB]
        idxd = rest[3 + _NB:3 + 2 * _NB]
        sems_g = rest[3 + 2 * _NB:3 + 3 * _NB]
        sems_i = rest[3 + 3 * _NB:3 + 4 * _NB]
        sems_s = rest[3 + 4 * _NB:3 + 5 * _NB]
        zsem = rest[3 + 5 * _NB]
        c = lax.axis_index("c")
        s = lax.axis_index("s")
        base = s * SR
        ebase = (c * _NS + s) * EPT

        pltpu.async_copy(src_hbm.at[pl.ds(ebase, EPT)], idx_s, zsem)

        def zrow(i, _):
            def zcol(j, _):
                zbuf[i, pl.ds(j * 16, 16)] = jnp.zeros((16,), jnp.float32)
                return 0
            return lax.fori_loop(0, H // 16, zcol, 0)
        lax.fori_loop(0, 16, zrow, 0)
        if with_deg:
            def z1(k, _):
                zb1[pl.ds(k * 16, 16)] = jnp.zeros((16,), jnp.float32)
                return 0
            lax.fori_loop(0, (SR + 16) // 16, z1, 0)

            def o1(k, _):
                ones[pl.ds(k * 16, 16)] = jnp.full((16,), 1.0, jnp.float32)
                return 0
            lax.fori_loop(0, (_C + 15) // 16, o1, 0)
        pltpu.make_async_copy(src_hbm.at[pl.ds(ebase, EPT)], idx_s,
                              zsem).wait()

        def zacc(k, _):
            pltpu.async_copy(zbuf, acc_sh.at[pl.ds(base + k * 16, 16)], zsem)
            return 0
        lax.fori_loop(0, NZ, zacc, 0)

        @pl.when(s == _NS - 1)
        def _():
            for t in range(EXTRA // 16):
                pltpu.async_copy(zbuf,
                                 acc_sh.at[pl.ds(SR * _NS + t * 16, 16)], zsem)
        if with_deg:
            pltpu.async_copy(zb1.at[pl.ds(0, SR)],
                             deg_sh.at[pl.ds(base, SR)], zsem)

            @pl.when(s == _NS - 1)
            def _():
                pltpu.async_copy(zb1.at[pl.ds(0, EXTRA)],
                                 deg_sh.at[pl.ds(SR * _NS, EXTRA)], zsem)

        def fire(j, b):
            pltpu.async_copy(dst_hbm.at[pl.ds(ebase + j * _C, _C)],
                             idxd[b], sems_i[b])
            pltpu.async_copy(h_hbm.at[idx_s.at[pl.ds(j * _C, _C)]],
                             rows[b], sems_g[b])

        def drain(j, b):
            pltpu.make_async_copy(dst_hbm.at[pl.ds(0, _C)],
                                  idxd[b], sems_i[b]).wait()
            pltpu.make_async_copy(h_hbm.at[idx_s.at[pl.ds(0, _C)]],
                                  rows[b], sems_g[b]).wait()

        def fire_scatter(j, b):
            pltpu.async_copy(rows[b], acc_sh.at[idxd[b]], sems_s[b],
                             add=True)
            if with_deg:
                pltpu.sync_copy(ones.at[pl.ds(0, _C)],
                                deg_sh.at[idxd[b]], add=True)

        def wait_scatter(j, b):
            pltpu.make_async_copy(rows[b], acc_sh.at[idxd[b]],
                                  sems_s[b]).wait()

        for b in range(_NB):
            fire(b, b)

        def zdrain(k, _):
            pltpu.make_async_copy(zbuf, acc_sh.at[pl.ds(base, 16)],
                                  zsem).wait()
            return 0
        lax.fori_loop(0, NZ, zdrain, 0)

        @pl.when(s == _NS - 1)
        def _():
            for t in range(EXTRA // 16):
                pltpu.make_async_copy(zbuf, acc_sh.at[pl.ds(base, 16)],
                                      zsem).wait()
        if with_deg:
            pltpu.make_async_copy(zb1.at[pl.ds(0, SR)],
                                  deg_sh.at[pl.ds(base, SR)], zsem).wait()

            @pl.when(s == _NS - 1)
            def _():
                pltpu.make_async_copy(zb1.at[pl.ds(0, EXTRA)],
                                      deg_sh.at[pl.ds(SR * _NS, EXTRA)],
                                      zsem).wait()

        plsc.subcore_barrier()

        def outer(o, _):
            g = o * _NB
            for b in range(_NB):
                drain(g + b, b)
                fire_scatter(g + b, b)
                wait_scatter(g + b, b)
                fire(g + _NB + b, b)
            return 0
        lax.fori_loop(0, NCH // _NB - 1, outer, 0)
        g_last = NCH - _NB
        for b in range(_NB):
            drain(g_last + b, b)
            fire_scatter(g_last + b, b)
            wait_scatter(g_last + b, b)

        plsc.subcore_barrier()

        def wb(out, deg):
            pltpu.sync_copy(acc_sh.at[pl.ds(base, SR)], out.at[pl.ds(base, SR)])
            if with_deg:
                pltpu.sync_copy(deg_sh.at[pl.ds(base, SR)],
                                zb1.at[pl.ds(0, SR)])
                pltpu.sync_copy(zb1.at[pl.ds(0, SR)],
                                deg.at[pl.ds(base, SR)])

            @pl.when(s == _NS - 1)
            def _():
                pltpu.sync_copy(acc_sh.at[pl.ds(SR * _NS, EXTRA)],
                                out.at[pl.ds(SR * _NS, EXTRA)])
                if with_deg:
                    pltpu.sync_copy(deg_sh.at[pl.ds(SR * _NS, EXTRA)],
                                    zb1.at[pl.ds(0, EXTRA)])
                    pltpu.sync_copy(zb1.at[pl.ds(0, EXTRA)],
                                    deg.at[pl.ds(SR * _NS, EXTRA)])

        @pl.when(c == 0)
        def _():
            wb(out0, deg0 if with_deg else None)

        @pl.when(c == 1)
        def _():
            wb(out1, deg1 if with_deg else None)

    return pl.kernel(body, out_type=out_type, mesh=mesh, scratch_types=scratch)


def _sc_scatter(h, src, dst):
    f = _sc_scatter_fn(h.shape[0], h.shape[1], src.shape[0], False)
    return f(h, src, dst)


def _sc_scatter_deg(h, src, dst):
    f = _sc_scatter_fn(h.shape[0], h.shape[1], src.shape[0], True)
    return f(h, src, dst)


def _row_spec(H):
    return pl.BlockSpec((_BLK, H), lambda i: (i, 0))


def _full_spec(a, b):
    return pl.BlockSpec((a, b), lambda i: (0, 0))


def _tc_graphconv1(a0, a1, x, W_rel, W_root, b, d0, d1):
    N, H = x.shape

    def body(a0r, a1r, xr, wr, wt, br, d0r, d1r, outr, disr):
        agg = a0r[...] + a1r[...]
        y = (jnp.dot(agg, wr[...], preferred_element_type=jnp.float32)
             + jnp.dot(xr[...], wt[...], preferred_element_type=jnp.float32)
             + br[...])
        outr[...] = jnp.maximum(y, 0.0)
        disr[...] = lax.rsqrt(d0r[...] + d1r[...] + 1.0)

    return pl.pallas_call(
        body,
        grid=(N // _BLK,),
        in_specs=([_row_spec(H)] * 3 + [_full_spec(H, H)] * 2
                  + [_full_spec(1, H)]
                  + [pl.BlockSpec((_BLK, 1), lambda i: (i, 0))] * 2),
        out_specs=[_row_spec(H), pl.BlockSpec((_BLK, 1), lambda i: (i, 0))],
        out_shape=[jax.ShapeDtypeStruct((N, H), jnp.float32),
                   jax.ShapeDtypeStruct((N, 1), jnp.float32)],
    )(a0, a1, x, W_rel, W_root, b.reshape(1, H), d0, d1)


def _tc_graphconv2_proj(a0, a1, h1, W_rel, W_root, b, W3, dis):
    N, H = h1.shape

    def body(a0r, a1r, hr, wr, wt, br, w3r, disr, outr):
        agg = a0r[...] + a1r[...]
        h2 = jnp.maximum(
            jnp.dot(agg, wr[...], preferred_element_type=jnp.float32)
            + jnp.dot(hr[...], wt[...], preferred_element_type=jnp.float32)
            + br[...], 0.0)
        outr[...] = disr[...] * jnp.dot(h2, w3r[...],
                                        preferred_element_type=jnp.float32)

    return pl.pallas_call(
        body,
        grid=(N // _BLK,),
        in_specs=([_row_spec(H)] * 3 + [_full_spec(H, H)] * 2
                  + [_full_spec(1, H), _full_spec(H, H)]
                  + [pl.BlockSpec((_BLK, 1), lambda i: (i, 0))]),
        out_specs=_row_spec(H),
        out_shape=jax.ShapeDtypeStruct((N, H), jnp.float32),
    )(a0, a1, h1, W_rel, W_root, b.reshape(1, H), W3, dis)


def _tc_gcn_post_proj(a0, a1, p, b, Wn, dis):
    N, H = p.shape

    def body(a0r, a1r, pr, br, wnr, disr, outr):
        dis_b = disr[...]
        h = jnp.maximum(dis_b * (a0r[...] + a1r[...] + pr[...]) + br[...], 0.0)
        outr[...] = dis_b * jnp.dot(h, wnr[...],
                                    preferred_element_type=jnp.float32)

    return pl.pallas_call(
        body,
        grid=(N // _BLK,),
        in_specs=([_row_spec(H)] * 3
                  + [_full_spec(1, H), _full_spec(H, H)]
                  + [pl.BlockSpec((_BLK, 1), lambda i: (i, 0))]),
        out_specs=_row_spec(H),
        out_shape=jax.ShapeDtypeStruct((N, H), jnp.float32),
    )(a0, a1, p, b.reshape(1, H), Wn, dis)


def _tc_final(a0, a1, p, b, dis, batch3d, W_lin, b_lin2d):
    N, H = p.shape
    OUT = W_lin.shape[1]

    def body(a0r, a1r, pr, br, disr, batr, wlr, blr,
             logits_ref, emb_ref, sums_sc, cnt_sc):
        i = pl.program_id(0)
        h5 = disr[...] * (a0r[...] + a1r[...] + pr[...]) + br[...]
        oh = (lax.broadcasted_iota(jnp.int32, (_G, _BLK), 0)
              == jnp.broadcast_to(batr[...].reshape(1, _BLK),
                                  (_G, _BLK))).astype(jnp.float32)
        psum = jnp.dot(oh, h5, preferred_element_type=jnp.float32)
        pcnt = jnp.sum(oh, axis=1, keepdims=True)

        @pl.when(i == 0)
        def _():
            sums_sc[...] = psum
            cnt_sc[...] = pcnt

        @pl.when(i > 0)
        def _():
            sums_sc[...] += psum
            cnt_sc[...] += pcnt

        @pl.when(i == pl.num_programs(0) - 1)
        def _():
            emb = sums_sc[...] / jnp.maximum(cnt_sc[...], 1.0)
            emb_ref[...] = emb
            logits_ref[...] = (jnp.dot(emb, wlr[...],
                                       preferred_element_type=jnp.float32)
                               + blr[...])

    return pl.pallas_call(
        body,
        grid=(N // _BLK,),
        in_specs=([_row_spec(H)] * 3
                  + [_full_spec(1, H)]
                  + [pl.BlockSpec((_BLK, 1), lambda i: (i, 0))]
                  + [pl.BlockSpec((1, 1, _BLK), lambda i: (i, 0, 0))]
                  + [_full_spec(H, OUT), _full_spec(1, OUT)]),
        out_specs=[pl.BlockSpec((_G, OUT), lambda i: (0, 0)),
                   pl.BlockSpec((_G, H), lambda i: (0, 0))],
        out_shape=[jax.ShapeDtypeStruct((_G, OUT), jnp.float32),
                   jax.ShapeDtypeStruct((_G, H), jnp.float32)],
        scratch_shapes=[pltpu.VMEM((_G, H), jnp.float32),
                        pltpu.VMEM((_G, 1), jnp.float32)],
    )(a0, a1, p, b.reshape(1, H), dis, batch3d, W_lin, b_lin2d)


def kernel(x, edge_index, batch, W_rel1, W_root1, b1, W_rel2, W_root2, b2,
           W3, b3, W4, b4, W5, b5, W_lin, b_lin):
    N, _ = x.shape
    src = edge_index[0]
    dst = edge_index[1]

    a10, a11, dg0, dg1 = _sc_scatter_deg(x, src, dst)
    h1, dis = _tc_graphconv1(a10, a11, x, W_rel1, W_root1, b1,
                             dg0.reshape(N, 1), dg1.reshape(N, 1))

    a20, a21 = _sc_scatter(h1, src, dst)
    p3 = _tc_graphconv2_proj(a20, a21, h1, W_rel2, W_root2, b2, W3, dis)

    a30, a31 = _sc_scatter(p3, src, dst)
    p4 = _tc_gcn_post_proj(a30, a31, p3, b3, W4, dis)
    a40, a41 = _sc_scatter(p4, src, dst)
    p5 = _tc_gcn_post_proj(a40, a41, p4, b4, W5, dis)

    a50, a51 = _sc_scatter(p5, src, dst)
    logits, embedding = _tc_final(a50, a51, p5, b5, dis,
                                  batch.reshape(N // _BLK, 1, _BLK), W_lin,
                                  b_lin.reshape(1, W_lin.shape[1]))
    return (logits, embedding)


# speedup vs baseline: 1.0236x; 1.0236x over previous
import functools

import jax
import jax.numpy as jnp
from jax import lax
from jax.experimental import pallas as pl
from jax.experimental.pallas import tpu as pltpu
from jax.experimental.pallas import tpu_sc as plsc

_NC = 2
_NS = 16
_G = 128
_BLK = 2000


_C = 40
_NB = 5


@functools.lru_cache(maxsize=None)
def _sc_scatter_fn(N, H, E, with_deg):
    NT = _NC * _NS
    EPT = E // NT
    assert E % NT == 0 and EPT % _C == 0
    NCH = EPT // _C
    assert NCH % _NB == 0
    SR = (N // _NS) // 8 * 8
    EXTRA = N - SR * _NS
    assert EXTRA % 16 == 0 and SR % 16 == 0
    NZ = SR // 16

    mesh = plsc.VectorSubcoreMesh(core_axis_name="c", subcore_axis_name="s",
                                  num_cores=_NC, num_subcores=_NS)

    out_type = [jax.ShapeDtypeStruct((N, H), jnp.float32),
                jax.ShapeDtypeStruct((N, H), jnp.float32)]
    scratch = [
        pltpu.VMEM_SHARED((N, H), jnp.float32),
        pltpu.VMEM((EPT,), jnp.int32),
        pltpu.VMEM((16, H), jnp.float32),
    ] + [pltpu.VMEM((_C, H), jnp.float32) for _ in range(_NB)] \
      + [pltpu.VMEM((_C,), jnp.int32) for _ in range(_NB)] \
      + [pltpu.SemaphoreType.DMA for _ in range(3 * _NB + 1)]
    if with_deg:
        out_type += [jax.ShapeDtypeStruct((N,), jnp.float32),
                     jax.ShapeDtypeStruct((N,), jnp.float32)]
        scratch += [
            pltpu.VMEM_SHARED((N,), jnp.float32),
            pltpu.VMEM((SR + 16,), jnp.float32),
            pltpu.VMEM((((_C + 15) // 16) * 16,), jnp.float32),
        ]

    def body(h_hbm, src_hbm, dst_hbm, out0, out1, *rest):
        if with_deg:
            deg0, deg1 = rest[0], rest[1]
            rest = rest[2:]
            deg_sh, zb1, ones = rest[-3:]
        acc_sh, idx_s, zbuf = rest[0], rest[1], rest[2]
        rows = rest[3:3 + _NB]
        idxd = rest[3 + _NB:3 + 2 * _NB]
        sems_g = rest[3 + 2 * _NB:3 + 3 * _NB]
        sems_i = rest[3 + 3 * _NB:3 + 4 * _NB]
        sems_s = rest[3 + 4 * _NB:3 + 5 * _NB]
        zsem = rest[3 + 5 * _NB]
        c = lax.axis_index("c")
        s = lax.axis_index("s")
        base = s * SR
        ebase = (c * _NS + s) * EPT

        pltpu.async_copy(src_hbm.at[pl.ds(ebase, EPT)], idx_s, zsem)

        def zrow(i, _):
            def zcol(j, _):
                zbuf[i, pl.ds(j * 16, 16)] = jnp.zeros((16,), jnp.float32)
                return 0
            return lax.fori_loop(0, H // 16, zcol, 0)
        lax.fori_loop(0, 16, zrow, 0)
        if with_deg:
            def z1(k, _):
                zb1[pl.ds(k * 16, 16)] = jnp.zeros((16,), jnp.float32)
                return 0
            lax.fori_loop(0, (SR + 16) // 16, z1, 0)

            def o1(k, _):
                ones[pl.ds(k * 16, 16)] = jnp.full((16,), 1.0, jnp.float32)
                return 0
            lax.fori_loop(0, (_C + 15) // 16, o1, 0)
        pltpu.make_async_copy(src_hbm.at[pl.ds(ebase, EPT)], idx_s,
                              zsem).wait()

        def zacc(k, _):
            pltpu.async_copy(zbuf, acc_sh.at[pl.ds(base + k * 16, 16)], zsem)
            return 0
        lax.fori_loop(0, NZ, zacc, 0)

        @pl.when(s == _NS - 1)
        def _():
            for t in range(EXTRA // 16):
                pltpu.async_copy(zbuf,
                                 acc_sh.at[pl.ds(SR * _NS + t * 16, 16)], zsem)
        if with_deg:
            pltpu.async_copy(zb1.at[pl.ds(0, SR)],
                             deg_sh.at[pl.ds(base, SR)], zsem)

            @pl.when(s == _NS - 1)
            def _():
                pltpu.async_copy(zb1.at[pl.ds(0, EXTRA)],
                                 deg_sh.at[pl.ds(SR * _NS, EXTRA)], zsem)

        def fire(j, b):
            pltpu.async_copy(dst_hbm.at[pl.ds(ebase + j * _C, _C)],
                             idxd[b], sems_i[b])
            pltpu.async_copy(h_hbm.at[idx_s.at[pl.ds(j * _C, _C)]],
                             rows[b], sems_g[b])

        def drain(j, b):
            pltpu.make_async_copy(dst_hbm.at[pl.ds(0, _C)],
                                  idxd[b], sems_i[b]).wait()
            pltpu.make_async_copy(h_hbm.at[idx_s.at[pl.ds(0, _C)]],
                                  rows[b], sems_g[b]).wait()

        def fire_scatter(j, b):
            pltpu.async_copy(rows[b], acc_sh.at[idxd[b]], sems_s[b],
                             add=True)
            if with_deg:
                pltpu.sync_copy(ones.at[pl.ds(0, _C)],
                                deg_sh.at[idxd[b]], add=True)

        def wait_scatter(j, b):
            pltpu.make_async_copy(rows[b], acc_sh.at[idxd[b]],
                                  sems_s[b]).wait()

        for b in range(_NB):
            fire(b, b)

        def zdrain(k, _):
            pltpu.make_async_copy(zbuf, acc_sh.at[pl.ds(base, 16)],
                                  zsem).wait()
            return 0
        lax.fori_loop(0, NZ, zdrain, 0)

        @pl.when(s == _NS - 1)
        def _():
            for t in range(EXTRA // 16):
                pltpu.make_async_copy(zbuf, acc_sh.at[pl.ds(base, 16)],
                                      zsem).wait()
        if with_deg:
            pltpu.make_async_copy(zb1.at[pl.ds(0, SR)],
                                  deg_sh.at[pl.ds(base, SR)], zsem).wait()

            @pl.when(s == _NS - 1)
            def _():
                pltpu.make_async_copy(zb1.at[pl.ds(0, EXTRA)],
                                      deg_sh.at[pl.ds(SR * _NS, EXTRA)],
                                      zsem).wait()

        plsc.subcore_barrier()

        def outer(o, _):
            g = o * _NB
            for b in range(_NB):
                drain(g + b, b)
                fire_scatter(g + b, b)
                wait_scatter(g + b, b)
                fire(g + _NB + b, b)
            return 0
        lax.fori_loop(0, NCH // _NB - 1, outer, 0)
        g_last = NCH - _NB
        for b in range(_NB):
            drain(g_last + b, b)
            fire_scatter(g_last + b, b)
            wait_scatter(g_last + b, b)

        plsc.subcore_barrier()

        def wb(out, deg):
            pltpu.sync_copy(acc_sh.at[pl.ds(base, SR)], out.at[pl.ds(base, SR)])
            if with_deg:
                pltpu.sync_copy(deg_sh.at[pl.ds(base, SR)],
                                zb1.at[pl.ds(0, SR)])
                pltpu.sync_copy(zb1.at[pl.ds(0, SR)],
                                deg.at[pl.ds(base, SR)])

            @pl.when(s == _NS - 1)
            def _():
                pltpu.sync_copy(acc_sh.at[pl.ds(SR * _NS, EXTRA)],
                                out.at[pl.ds(SR * _NS, EXTRA)])
                if with_deg:
                    pltpu.sync_copy(deg_sh.at[pl.ds(SR * _NS, EXTRA)],
                                    zb1.at[pl.ds(0, EXTRA)])
                    pltpu.sync_copy(zb1.at[pl.ds(0, EXTRA)],
                                    deg.at[pl.ds(SR * _NS, EXTRA)])

        @pl.when(c == 0)
        def _():
            wb(out0, deg0 if with_deg else None)

        @pl.when(c == 1)
        def _():
            wb(out1, deg1 if with_deg else None)

    return pl.kernel(body, out_type=out_type, mesh=mesh, scratch_types=scratch)


def _sc_scatter(h, src, dst):
    f = _sc_scatter_fn(h.shape[0], h.shape[1], src.shape[0], False)
    return f(h, src, dst)


def _sc_scatter_deg(h, src, dst):
    f = _sc_scatter_fn(h.shape[0], h.shape[1], src.shape[0], True)
    return f(h, src, dst)


def _row_spec(H):
    return pl.BlockSpec((_BLK, H), lambda i: (i, 0))


def _full_spec(a, b):
    return pl.BlockSpec((a, b), lambda i: (0, 0))


def _tc_graphconv1(a0, a1, x, W_rel, W_root, b, d0, d1):
    N, H = x.shape

    def body(a0r, a1r, xr, wr, wt, br, d0r, d1r, outr, disr):
        agg = a0r[...] + a1r[...]
        y = (jnp.dot(agg, wr[...], preferred_element_type=jnp.float32)
             + jnp.dot(xr[...], wt[...], preferred_element_type=jnp.float32)
             + br[...])
        outr[...] = jnp.maximum(y, 0.0)
        disr[...] = lax.rsqrt(d0r[...] + d1r[...] + 1.0)

    return pl.pallas_call(
        body,
        grid=(N // _BLK,),
        in_specs=([_row_spec(H)] * 3 + [_full_spec(H, H)] * 2
                  + [_full_spec(1, H)]
                  + [pl.BlockSpec((1, 1, _BLK), lambda i: (i, 0, 0))] * 2),
        out_specs=[_row_spec(H),
                   pl.BlockSpec((1, 1, _BLK), lambda i: (i, 0, 0))],
        out_shape=[jax.ShapeDtypeStruct((N, H), jnp.float32),
                   jax.ShapeDtypeStruct((N // _BLK, 1, _BLK), jnp.float32)],
    )(a0, a1, x, W_rel, W_root, b.reshape(1, H), d0, d1)


def _tc_graphconv2_proj(a0, a1, h1, W_rel, W_root, b, W3, dis):
    N, H = h1.shape

    def body(a0r, a1r, hr, wr, wt, br, w3r, disr, outr):
        agg = a0r[...] + a1r[...]
        h2 = jnp.maximum(
            jnp.dot(agg, wr[...], preferred_element_type=jnp.float32)
            + jnp.dot(hr[...], wt[...], preferred_element_type=jnp.float32)
            + br[...], 0.0)
        outr[...] = (jnp.transpose(disr[...].reshape(1, _BLK))
                     * jnp.dot(h2, w3r[...],
                               preferred_element_type=jnp.float32))

    return pl.pallas_call(
        body,
        grid=(N // _BLK,),
        in_specs=([_row_spec(H)] * 3 + [_full_spec(H, H)] * 2
                  + [_full_spec(1, H), _full_spec(H, H)]
                  + [pl.BlockSpec((1, 1, _BLK), lambda i: (i, 0, 0))]),
        out_specs=_row_spec(H),
        out_shape=jax.ShapeDtypeStruct((N, H), jnp.float32),
    )(a0, a1, h1, W_rel, W_root, b.reshape(1, H), W3, dis)


def _tc_gcn_post_proj(a0, a1, p, b, Wn, dis):
    N, H = p.shape

    def body(a0r, a1r, pr, br, wnr, disr, outr):
        dis_b = jnp.transpose(disr[...].reshape(1, _BLK))
        h = jnp.maximum(dis_b * (a0r[...] + a1r[...] + pr[...]) + br[...], 0.0)
        outr[...] = dis_b * jnp.dot(h, wnr[...],
                                    preferred_element_type=jnp.float32)

    return pl.pallas_call(
        body,
        grid=(N // _BLK,),
        in_specs=([_row_spec(H)] * 3
                  + [_full_spec(1, H), _full_spec(H, H)]
                  + [pl.BlockSpec((1, 1, _BLK), lambda i: (i, 0, 0))]),
        out_specs=_row_spec(H),
        out_shape=jax.ShapeDtypeStruct((N, H), jnp.float32),
    )(a0, a1, p, b.reshape(1, H), Wn, dis)


def _tc_final(a0, a1, p, b, dis, batch3d, W_lin, b_lin2d):
    N, H = p.shape
    OUT = W_lin.shape[1]

    def body(a0r, a1r, pr, br, disr, batr, wlr, blr,
             logits_ref, emb_ref, sums_sc, cnt_sc):
        i = pl.program_id(0)
        h5 = (jnp.transpose(disr[...].reshape(1, _BLK))
              * (a0r[...] + a1r[...] + pr[...]) + br[...])
        oh = (lax.broadcasted_iota(jnp.int32, (_G, _BLK), 0)
              == jnp.broadcast_to(batr[...].reshape(1, _BLK),
                                  (_G, _BLK))).astype(jnp.float32)
        psum = jnp.dot(oh, h5, preferred_element_type=jnp.float32)
        pcnt = jnp.sum(oh, axis=1, keepdims=True)

        @pl.when(i == 0)
        def _():
            sums_sc[...] = psum
            cnt_sc[...] = pcnt

        @pl.when(i > 0)
        def _():
            sums_sc[...] += psum
            cnt_sc[...] += pcnt

        @pl.when(i == pl.num_programs(0) - 1)
        def _():
            emb = sums_sc[...] / jnp.maximum(cnt_sc[...], 1.0)
            emb_ref[...] = emb
            logits_ref[...] = (jnp.dot(emb, wlr[...],
                                       preferred_element_type=jnp.float32)
                               + blr[...])

    return pl.pallas_call(
        body,
        grid=(N // _BLK,),
        in_specs=([_row_spec(H)] * 3
                  + [_full_spec(1, H)]
                  + [pl.BlockSpec((1, 1, _BLK), lambda i: (i, 0, 0))]
                  + [pl.BlockSpec((1, 1, _BLK), lambda i: (i, 0, 0))]
                  + [_full_spec(H, OUT), _full_spec(1, OUT)]),
        out_specs=[pl.BlockSpec((_G, OUT), lambda i: (0, 0)),
                   pl.BlockSpec((_G, H), lambda i: (0, 0))],
        out_shape=[jax.ShapeDtypeStruct((_G, OUT), jnp.float32),
                   jax.ShapeDtypeStruct((_G, H), jnp.float32)],
        scratch_shapes=[pltpu.VMEM((_G, H), jnp.float32),
                        pltpu.VMEM((_G, 1), jnp.float32)],
    )(a0, a1, p, b.reshape(1, H), dis, batch3d, W_lin, b_lin2d)


def kernel(x, edge_index, batch, W_rel1, W_root1, b1, W_rel2, W_root2, b2,
           W3, b3, W4, b4, W5, b5, W_lin, b_lin):
    N, _ = x.shape
    src = edge_index[0]
    dst = edge_index[1]

    a10, a11, dg0, dg1 = _sc_scatter_deg(x, src, dst)
    h1, dis = _tc_graphconv1(a10, a11, x, W_rel1, W_root1, b1,
                             dg0.reshape(N // _BLK, 1, _BLK),
                             dg1.reshape(N // _BLK, 1, _BLK))

    a20, a21 = _sc_scatter(h1, src, dst)
    p3 = _tc_graphconv2_proj(a20, a21, h1, W_rel2, W_root2, b2, W3, dis)

    a30, a31 = _sc_scatter(p3, src, dst)
    p4 = _tc_gcn_post_proj(a30, a31, p3, b3, W4, dis)
    a40, a41 = _sc_scatter(p4, src, dst)
    p5 = _tc_gcn_post_proj(a40, a41, p4, b4, W5, dis)

    a50, a51 = _sc_scatter(p5, src, dst)
    logits, embedding = _tc_final(a50, a51, p5, b5, dis,
                                  batch.reshape(N // _BLK, 1, _BLK), W_lin,
                                  b_lin.reshape(1, W_lin.shape[1]))
    return (logits, embedding)
